# baseline (speedup 1.0000x reference)
import functools

import jax
import jax.numpy as jnp
from jax import lax
from jax.experimental import pallas as pl
from jax.experimental.pallas import tpu as pltpu

try:
    from jax.experimental.pallas import tpu_sc as plsc
    _INFO = plsc.get_sparse_core_info()
    _NC, _NS = _INFO.num_cores, _INFO.num_subcores
except Exception:
    plsc = None
    _NC, _NS = 2, 16

_NW = _NC * _NS
_S = 8192
_D = 768
_EPS = 1e-12
_CH = 32
_NBUF = 5
_LOOK = 3
_BS = 1024


def _sc_gather(word_table, idx3):
    nchunk = idx3.shape[1]
    b_per_w = nchunk * _CH
    rows = _NW * b_per_w
    mesh = plsc.VectorSubcoreMesh(core_axis_name="c", subcore_axis_name="s")

    @functools.partial(
        pl.kernel,
        mesh=mesh,
        out_type=jax.ShapeDtypeStruct((rows, _D), jnp.float32),
        scratch_types=(
            [pltpu.VMEM((nchunk, _CH), jnp.int32)]
            + [pltpu.VMEM((_CH, _D), jnp.float32) for _ in range(_NBUF)]
            + [pltpu.SemaphoreType.DMA for _ in range(2 * _NBUF)]
        ),
    )
    def k(table_hbm, idx_hbm, out_hbm, idx_v, *bufsems):
        bufs = bufsems[:_NBUF]
        gsems = bufsems[_NBUF:2 * _NBUF]
        wsems = bufsems[2 * _NBUF:]
        wid = lax.axis_index("s") * _NC + lax.axis_index("c")
        base = wid * b_per_w
        pltpu.sync_copy(idx_hbm.at[wid], idx_v)
        gcopies = [None] * nchunk
        wcopies = [None] * nchunk
        for c in range(min(_LOOK, nchunk)):
            gcopies[c] = pltpu.async_copy(
                table_hbm.at[idx_v.at[c]], bufs[c % _NBUF], gsems[c % _NBUF])
        for c in range(nchunk):
            gcopies[c].wait()
            wcopies[c] = pltpu.async_copy(
                bufs[c % _NBUF], out_hbm.at[pl.ds(base + c * _CH, _CH)],
                wsems[c % _NBUF])
            n = c + _LOOK
            if n < nchunk:
                prev = n - _NBUF
                if prev >= 0:
                    wcopies[prev].wait()
                gcopies[n] = pltpu.async_copy(
                    table_hbm.at[idx_v.at[n]], bufs[n % _NBUF],
                    gsems[n % _NBUF])
        for c in range(max(0, nchunk - _NBUF), nchunk):
            wcopies[c].wait()

    return k(word_table, idx3)


def _tc_body(gl_ref, gh_ref, pl_ref, ph_ref, tt_ref, tte_ref, gam_ref, bet_ref,
             w_ref, b_ref, o_ref):
    x = jnp.concatenate(
        [gl_ref[...] + pl_ref[...], gh_ref[...] + ph_ref[...]], axis=1)
    tt = tt_ref[0]
    t0 = tte_ref[0:1, :]
    t1 = tte_ref[1:2, :]
    x = x + jnp.where(tt == 0, t0, t1)
    mu = jnp.mean(x, axis=1, keepdims=True)
    xc = x - mu
    var = jnp.mean(xc * xc, axis=1, keepdims=True)
    xn = xc * lax.rsqrt(var + _EPS)
    xn = xn * gam_ref[...] + bet_ref[...]
    y = lax.dot_general(xn.astype(jnp.bfloat16), w_ref[...],
                        (((1,), (1,)), ((), ())),
                        preferred_element_type=jnp.float32)
    o_ref[...] = y + b_ref[...]


def _tc_call(gathered, pos_slice, tt3, type_table, gamma2, beta2, W, b2):
    rows = gathered.shape[0]
    grid = rows // _BS
    return pl.pallas_call(
        _tc_body,
        grid=(grid,),
        in_specs=[
            pl.BlockSpec((_BS, _D // 2), lambda i: (i, 0)),
            pl.BlockSpec((_BS, _D // 2), lambda i: (i, 1)),
            pl.BlockSpec((_BS, _D // 2), lambda i: (i, 0)),
            pl.BlockSpec((_BS, _D // 2), lambda i: (i, 1)),
            pl.BlockSpec((1, _BS, 1), lambda i: (i, 0, 0)),
            pl.BlockSpec((2, _D), lambda i: (0, 0)),
            pl.BlockSpec((1, _D), lambda i: (0, 0)),
            pl.BlockSpec((1, _D), lambda i: (0, 0)),
            pl.BlockSpec((_D, _D), lambda i: (0, 0)),
            pl.BlockSpec((1, _D), lambda i: (0, 0)),
        ],
        out_specs=pl.BlockSpec((_BS, _D), lambda i: (i, 0)),
        out_shape=jax.ShapeDtypeStruct((rows, _D), jnp.float32),
    )(gathered, gathered, pos_slice, pos_slice, tt3, type_table,
      gamma2, beta2, W, b2)


def kernel(input_ids, token_type_ids, word_table, pos_table, type_table,
           ln_gamma, ln_beta, W, b):
    ids = input_ids.astype(jnp.int32)
    tt = token_type_ids.astype(jnp.int32)
    idx3 = ids.reshape(_NW, -1, _CH)
    gathered = _sc_gather(word_table, idx3)
    tt3 = tt.reshape(_S // _BS, _BS, 1)
    out = _tc_call(gathered, pos_table[:_S], tt3, type_table,
                   ln_gamma.reshape(1, _D), ln_beta.reshape(1, _D),
                   W.astype(jnp.bfloat16), b.reshape(1, _D))
    return out.reshape(1, _S, _D)


# speedup vs baseline: 1.0051x; 1.0051x over previous
import functools

import jax
import jax.numpy as jnp
from jax import lax
from jax.experimental import pallas as pl
from jax.experimental.pallas import tpu as pltpu

try:
    from jax.experimental.pallas import tpu_sc as plsc
    _INFO = plsc.get_sparse_core_info()
    _NC, _NS = _INFO.num_cores, _INFO.num_subcores
except Exception:
    plsc = None
    _NC, _NS = 2, 16

_NW = _NC * _NS
_S = 8192
_D = 768
_EPS = 1e-12
_CH = 32
_NBUF = 5
_LOOK = 3
_BS = 1024


def _sc_gather(word_table, idx3):
    nchunk = idx3.shape[1]
    b_per_w = nchunk * _CH
    rows = _NW * b_per_w
    mesh = plsc.VectorSubcoreMesh(core_axis_name="c", subcore_axis_name="s")

    @functools.partial(
        pl.kernel,
        mesh=mesh,
        out_type=jax.ShapeDtypeStruct((rows, _D), jnp.float32),
        scratch_types=(
            [pltpu.VMEM((nchunk, _CH), jnp.int32)]
            + [pltpu.VMEM((_CH, _D), jnp.float32) for _ in range(_NBUF)]
            + [pltpu.SemaphoreType.DMA for _ in range(2 * _NBUF)]
        ),
    )
    def k(table_hbm, idx_hbm, out_hbm, idx_v, *bufsems):
        bufs = bufsems[:_NBUF]
        gsems = bufsems[_NBUF:2 * _NBUF]
        wsems = bufsems[2 * _NBUF:]
        wid = lax.axis_index("s") * _NC + lax.axis_index("c")
        base = wid * b_per_w
        pltpu.sync_copy(idx_hbm.at[wid], idx_v)
        gcopies = [None] * nchunk
        wcopies = [None] * nchunk
        for c in range(min(_LOOK, nchunk)):
            gcopies[c] = pltpu.async_copy(
                table_hbm.at[idx_v.at[c]], bufs[c % _NBUF], gsems[c % _NBUF])
        for c in range(nchunk):
            gcopies[c].wait()
            wcopies[c] = pltpu.async_copy(
                bufs[c % _NBUF], out_hbm.at[pl.ds(base + c * _CH, _CH)],
                wsems[c % _NBUF])
            n = c + _LOOK
            if n < nchunk:
                prev = n - _NBUF
                if prev >= 0:
                    wcopies[prev].wait()
                gcopies[n] = pltpu.async_copy(
                    table_hbm.at[idx_v.at[n]], bufs[n % _NBUF],
                    gsems[n % _NBUF])
        for c in range(max(0, nchunk - _NBUF), nchunk):
            wcopies[c].wait()

    return k(word_table, idx3)


def _tc_body(g_ref, pos_ref, tt_ref, tte_ref, gam_ref, bet_ref, w_ref, b_ref, o_ref):
    x = g_ref[...] + pos_ref[...]
    tt = tt_ref[0]
    t0 = tte_ref[0:1, :]
    t1 = tte_ref[1:2, :]
    x = x + jnp.where(tt == 0, t0, t1)
    mu = jnp.mean(x, axis=1, keepdims=True)
    xc = x - mu
    var = jnp.mean(xc * xc, axis=1, keepdims=True)
    xn = xc * lax.rsqrt(var + _EPS)
    xn = xn * gam_ref[...] + bet_ref[...]
    y = lax.dot_general(xn.astype(jnp.bfloat16),
                        w_ref[...].astype(jnp.bfloat16),
                        (((1,), (1,)), ((), ())),
                        preferred_element_type=jnp.float32)
    o_ref[...] = y + b_ref[...]


def _tc_call(gathered, pos_slice, tt3, type_table, gamma2, beta2, W, b2):
    rows = gathered.shape[0]
    grid = rows // _BS
    return pl.pallas_call(
        _tc_body,
        grid=(grid,),
        in_specs=[
            pl.BlockSpec((_BS, _D), lambda i: (i, 0)),
            pl.BlockSpec((_BS, _D), lambda i: (i, 0)),
            pl.BlockSpec((1, _BS, 1), lambda i: (i, 0, 0)),
            pl.BlockSpec((2, _D), lambda i: (0, 0)),
            pl.BlockSpec((1, _D), lambda i: (0, 0)),
            pl.BlockSpec((1, _D), lambda i: (0, 0)),
            pl.BlockSpec((_D, _D), lambda i: (0, 0)),
            pl.BlockSpec((1, _D), lambda i: (0, 0)),
        ],
        out_specs=pl.BlockSpec((_BS, _D), lambda i: (i, 0)),
        out_shape=jax.ShapeDtypeStruct((rows, _D), jnp.float32),
    )(gathered, pos_slice, tt3, type_table, gamma2, beta2, W, b2)


def kernel(input_ids, token_type_ids, word_table, pos_table, type_table,
           ln_gamma, ln_beta, W, b):
    ids = input_ids.astype(jnp.int32)
    tt = token_type_ids.astype(jnp.int32)
    idx3 = ids.reshape(_NW, -1, _CH)
    gathered = _sc_gather(word_table, idx3)
    tt3 = tt.reshape(_S // _BS, _BS, 1)
    out = _tc_call(gathered, pos_table[:_S], tt3, type_table,
                   ln_gamma.reshape(1, _D), ln_beta.reshape(1, _D),
                   W, b.reshape(1, _D))
    return out.reshape(1, _S, _D)
